# baseline (speedup 1.0000x reference)
import functools

import jax
import jax.numpy as jnp
from jax import lax
from jax.experimental import pallas as pl
from jax.experimental.pallas import tpu as pltpu
from jax.experimental.pallas import tpu_sc as plsc

_INFO = plsc.get_sparse_core_info()
_NC, _NS, _L = _INFO.num_cores, _INFO.num_subcores, _INFO.num_lanes
_NW = _NC * _NS
_CH = 128


def kernel(user, song, user_weight, song_weight):
    B = user.shape[0]
    D = user_weight.shape[1]
    b_per_w = B // _NW
    n_chunks = b_per_w // _CH
    n_groups = b_per_w // _L

    mesh = plsc.VectorSubcoreMesh(core_axis_name="c", subcore_axis_name="s")

    @functools.partial(
        pl.kernel,
        mesh=mesh,
        out_type=jax.ShapeDtypeStruct((B,), jnp.float32),
        compiler_params=pltpu.CompilerParams(
            needs_layout_passes=False, use_tc_tiling_on_sc=False),
        scratch_types=[
            pltpu.VMEM((n_chunks, _CH), jnp.int32),
            pltpu.VMEM((n_chunks, _CH), jnp.int32),
            pltpu.VMEM((b_per_w, D), jnp.float32),
            pltpu.VMEM((b_per_w, D), jnp.float32),
            pltpu.VMEM((b_per_w,), jnp.float32),
            pltpu.SemaphoreType.DMA,
        ],
    )
    def sc_kernel(user_hbm, song_hbm, uw_hbm, sw_hbm, out_hbm,
                  uidx, sidx, urows, srows, outv, sem):
        wid = lax.axis_index("s") * _NC + lax.axis_index("c")
        row0 = wid * n_chunks
        base = wid * b_per_w

        pltpu.sync_copy(user_hbm.at[pl.ds(row0, n_chunks)], uidx)
        pltpu.sync_copy(song_hbm.at[pl.ds(row0, n_chunks)], sidx)

        copies = []
        for j in range(n_chunks):
            copies.append(pltpu.async_copy(
                uw_hbm.at[uidx.at[j]], urows.at[pl.ds(j * _CH, _CH)], sem))
            copies.append(pltpu.async_copy(
                sw_hbm.at[sidx.at[j]], srows.at[pl.ds(j * _CH, _CH)], sem))
        for c in copies:
            c.wait()

        lane = lax.iota(jnp.int32, _L)

        def group_body(g, carry):
            base_row = g * _L
            acc = jnp.zeros((_L,), jnp.float32)
            for k in range(_L):
                u = urows[base_row + k]
                s = srows[base_row + k]
                t = jnp.sum(u * s)
                acc = jnp.where(lane == k, t, acc)
            outv[pl.ds(base_row, _L)] = jnp.maximum(acc, 0.0)
            return carry

        lax.fori_loop(0, n_groups, group_body, 0)

        pltpu.sync_copy(outv, out_hbm.at[pl.ds(base, b_per_w)])

    user2d = user.reshape(_NW * n_chunks, _CH)
    song2d = song.reshape(_NW * n_chunks, _CH)
    return sc_kernel(user2d, song2d, user_weight, song_weight)
